# Initial kernel scaffold; baseline (speedup 1.0000x reference)
import functools

import jax
import jax.numpy as jnp
from jax import lax
from jax.experimental import pallas as pl
from jax.experimental.pallas import tpu as pltpu
from jax.experimental.pallas import tpu_sc as plsc

N = 10000
E = 320000
C = 128
DE = 16
H = 64
NREP = 2

NC = 2
NS = 16
CH = C // NC
CHUNK = 128
NCHUNKS = E // CHUNK
BASE_CH = NCHUNKS // NS
EXTRA_CH = NCHUNKS - BASE_CH * NS
ROWS_PER_SUB = N // NS
NPAD = 10240
DROWS = NPAD // NS


_EB = 3200


def _mlp_body(ea_ref, w1_ref, b1_ref, w2_ref, b2_ref, w_ref):
    h = jnp.dot(ea_ref[...], w1_ref[...], preferred_element_type=jnp.float32)
    h = jnp.maximum(h + b1_ref[...], 0.0)
    w = jnp.dot(h, w2_ref[...], preferred_element_type=jnp.float32) + b2_ref[...]
    w_ref[0] = w[:, :CH]
    w_ref[1] = w[:, CH:]


_mlp = pl.pallas_call(
    _mlp_body,
    grid=(E // _EB,),
    in_specs=[
        pl.BlockSpec((_EB, DE), lambda i: (i, 0)),
        pl.BlockSpec((DE, H), lambda i: (0, 0)),
        pl.BlockSpec((1, H), lambda i: (0, 0)),
        pl.BlockSpec((H, C), lambda i: (0, 0)),
        pl.BlockSpec((1, C), lambda i: (0, 0)),
    ],
    out_specs=pl.BlockSpec((NC, _EB, CH), lambda i: (0, i, 0)),
    out_shape=jax.ShapeDtypeStruct((NC, E, CH), jnp.float32),
)

_RB = 2000


def _softmax_body(x_ref, q_ref):
    x = x_ref[...]
    m = jnp.max(x, axis=1, keepdims=True)
    e = jnp.exp(x - m)
    q = e / jnp.sum(e, axis=1, keepdims=True)
    q_ref[0] = q[:, :CH]
    q_ref[1] = q[:, CH:]


_softmax = pl.pallas_call(
    _softmax_body,
    grid=(N // _RB,),
    in_specs=[pl.BlockSpec((_RB, C), lambda i: (i, 0))],
    out_specs=pl.BlockSpec((NC, _RB, CH), lambda i: (0, i, 0)),
    out_shape=jax.ShapeDtypeStruct((NC, N, CH), jnp.float32),
)


def _upd_softmax_body(x_ref, agg_ref, deg_ref, q_ref):
    agg = jnp.concatenate([agg_ref[0], agg_ref[1]], axis=1)
    deg = jnp.maximum(deg_ref[...], 1.0)
    x = x_ref[...] - agg / deg
    m = jnp.max(x, axis=1, keepdims=True)
    e = jnp.exp(x - m)
    q = e / jnp.sum(e, axis=1, keepdims=True)
    q_ref[0] = q[:, :CH]
    q_ref[1] = q[:, CH:]


_upd_softmax = pl.pallas_call(
    _upd_softmax_body,
    grid=(N // _RB,),
    in_specs=[
        pl.BlockSpec((_RB, C), lambda i: (i, 0)),
        pl.BlockSpec((NC, _RB, CH), lambda i: (0, i, 0)),
        pl.BlockSpec((_RB, 1), lambda i: (i, 0)),
    ],
    out_specs=pl.BlockSpec((NC, _RB, CH), lambda i: (0, i, 0)),
    out_shape=jax.ShapeDtypeStruct((NC, N, CH), jnp.float32),
)


def _upd_final_body(x_ref, agg_ref, deg_ref, out_ref):
    agg = jnp.concatenate([agg_ref[0], agg_ref[1]], axis=1)
    deg = jnp.maximum(deg_ref[...], 1.0)
    out_ref[...] = x_ref[...] - agg / deg


_upd_final = pl.pallas_call(
    _upd_final_body,
    grid=(N // _RB,),
    in_specs=[
        pl.BlockSpec((_RB, C), lambda i: (i, 0)),
        pl.BlockSpec((NC, _RB, CH), lambda i: (0, i, 0)),
        pl.BlockSpec((_RB, 1), lambda i: (i, 0)),
    ],
    out_specs=pl.BlockSpec((_RB, C), lambda i: (i, 0)),
    out_shape=jax.ShapeDtypeStruct((N, C), jnp.float32),
)


_mesh = plsc.VectorSubcoreMesh(
    core_axis_name="c", subcore_axis_name="s", num_cores=NC, num_subcores=NS)


@functools.partial(
    pl.kernel,
    out_type=[
        jax.ShapeDtypeStruct((NC, N, CH), jnp.float32),
        jax.ShapeDtypeStruct((NC, NPAD), jnp.float32),
    ],
    mesh=_mesh,
    scratch_types=[
        pltpu.VMEM_SHARED((N, CH), jnp.float32),
        pltpu.VMEM_SHARED((N, CH), jnp.float32),
        pltpu.VMEM_SHARED((NPAD,), jnp.float32),
        pltpu.VMEM((CHUNK,), jnp.int32),
        pltpu.VMEM((CHUNK,), jnp.int32),
        pltpu.VMEM((CHUNK,), jnp.float32),
        pltpu.VMEM((CHUNK, CH), jnp.float32),
        pltpu.VMEM((CHUNK, CH), jnp.float32),
        pltpu.VMEM((CHUNK, CH), jnp.float32),
        pltpu.SemaphoreType.DMA,
    ],
)
def _sc_propagate(q_hbm, w_hbm, src_hbm, dst_hbm, zrow_hbm, zdeg_hbm,
                  agg_out, deg_out,
                  q_sh, agg_sh, deg_sh, src_v, dst_v, ones_v,
                  q_rows, w_rows, msg, sem):
    cid = lax.axis_index("c")
    sid = lax.axis_index("s")

    r0 = sid * ROWS_PER_SUB
    pltpu.sync_copy(q_hbm.at[cid, pl.ds(r0, ROWS_PER_SUB)],
                    q_sh.at[pl.ds(r0, ROWS_PER_SUB)])
    pltpu.sync_copy(zrow_hbm.at[pl.ds(r0, ROWS_PER_SUB)],
                    agg_sh.at[pl.ds(r0, ROWS_PER_SUB)])
    d0 = sid * DROWS
    pltpu.sync_copy(zdeg_hbm.at[pl.ds(d0, DROWS)], deg_sh.at[pl.ds(d0, DROWS)])
    for j in range(CHUNK // 16):
        ones_v[pl.ds(j * 16, 16)] = jnp.full((16,), 1.0, jnp.float32)
    plsc.subcore_barrier()

    nch = jnp.where(sid < EXTRA_CH, BASE_CH + 1, BASE_CH)
    c0 = sid * BASE_CH + jnp.minimum(sid, EXTRA_CH)

    def chunk_body(i, carry):
        base = (c0 + i) * CHUNK
        pltpu.sync_copy(src_hbm.at[pl.ds(base, CHUNK)], src_v)
        pltpu.sync_copy(dst_hbm.at[pl.ds(base, CHUNK)], dst_v)
        pltpu.sync_copy(w_hbm.at[cid, pl.ds(base, CHUNK)], w_rows)
        pltpu.async_copy(q_sh.at[src_v], q_rows, sem).wait()

        def mul_body(e, c2):
            for j in range(CH // 16):
                sl = pl.ds(j * 16, 16)
                msg[e, sl] = q_rows[e, sl] * w_rows[e, sl]
            return c2

        lax.fori_loop(0, CHUNK, mul_body, 0)
        pltpu.sync_copy(msg, agg_sh.at[dst_v], add=True)
        pltpu.sync_copy(ones_v, deg_sh.at[dst_v], add=True)
        return carry

    lax.fori_loop(0, nch, chunk_body, 0)
    plsc.subcore_barrier()

    pltpu.sync_copy(agg_sh.at[pl.ds(r0, ROWS_PER_SUB)],
                    agg_out.at[cid, pl.ds(r0, ROWS_PER_SUB)])
    pltpu.sync_copy(deg_sh.at[pl.ds(d0, DROWS)], deg_out.at[cid, pl.ds(d0, DROWS)])


def kernel(input, edge_index, edge_attr, W1, b1, W2, b2):
    src = edge_index[0]
    dst = edge_index[1]
    w = _mlp(edge_attr, W1, b1.reshape(1, H), W2, b2.reshape(1, C))
    zrow = jnp.zeros((N, CH), jnp.float32)
    zdeg = jnp.zeros((NPAD,), jnp.float32)

    q = _softmax(input)
    agg, deg = _sc_propagate(q, w, src, dst, zrow, zdeg)
    deg_col = deg[0, :N].reshape(N, 1)
    q = _upd_softmax(input, agg, deg_col)
    agg, _ = _sc_propagate(q, w, src, dst, zrow, zdeg)
    return _upd_final(input, agg, deg_col)


# speedup vs baseline: 3.2293x; 3.2293x over previous
import functools

import jax
import jax.numpy as jnp
from jax import lax
from jax.experimental import pallas as pl
from jax.experimental.pallas import tpu as pltpu
from jax.experimental.pallas import tpu_sc as plsc

N = 10000
E = 320000
C = 128
DE = 16
H = 64

NC = 2
NS = 16
CHUNK = 128
NCHUNKS = E // CHUNK
CORE_CHUNKS = NCHUNKS // NC
BASE_CH = CORE_CHUNKS // NS
EXTRA_CH = CORE_CHUNKS - BASE_CH * NS
RSTAGE = 624
RSTAGE_LAST = N - RSTAGE * (NS - 1)
NPAD = 10240
DROWS = NPAD // NS


_EB = 3200


def _mlp_body(ea_ref, w1_ref, b1_ref, w2_ref, b2_ref, w_ref):
    h = jnp.dot(ea_ref[...], w1_ref[...], preferred_element_type=jnp.float32)
    h = jnp.maximum(h + b1_ref[...], 0.0)
    w_ref[...] = jnp.dot(h, w2_ref[...], preferred_element_type=jnp.float32) + b2_ref[...]


_mlp = pl.pallas_call(
    _mlp_body,
    grid=(E // _EB,),
    in_specs=[
        pl.BlockSpec((_EB, DE), lambda i: (i, 0)),
        pl.BlockSpec((DE, H), lambda i: (0, 0)),
        pl.BlockSpec((1, H), lambda i: (0, 0)),
        pl.BlockSpec((H, C), lambda i: (0, 0)),
        pl.BlockSpec((1, C), lambda i: (0, 0)),
    ],
    out_specs=pl.BlockSpec((_EB, C), lambda i: (i, 0)),
    out_shape=jax.ShapeDtypeStruct((E, C), jnp.float32),
)

_RB = 2000


def _softmax_body(x_ref, q_ref):
    x = x_ref[...]
    m = jnp.max(x, axis=1, keepdims=True)
    e = jnp.exp(x - m)
    q_ref[...] = e / jnp.sum(e, axis=1, keepdims=True)


_softmax = pl.pallas_call(
    _softmax_body,
    grid=(N // _RB,),
    in_specs=[pl.BlockSpec((_RB, C), lambda i: (i, 0))],
    out_specs=pl.BlockSpec((_RB, C), lambda i: (i, 0)),
    out_shape=jax.ShapeDtypeStruct((N, C), jnp.float32),
)


def _upd_softmax_body(x_ref, agg_ref, d0_ref, d1_ref, q_ref):
    agg = agg_ref[0] + agg_ref[1]
    deg = jnp.maximum(d0_ref[...] + d1_ref[...], 1.0)
    x = x_ref[...] - agg / deg
    m = jnp.max(x, axis=1, keepdims=True)
    e = jnp.exp(x - m)
    q_ref[...] = e / jnp.sum(e, axis=1, keepdims=True)


_upd_softmax = pl.pallas_call(
    _upd_softmax_body,
    grid=(N // _RB,),
    in_specs=[
        pl.BlockSpec((_RB, C), lambda i: (i, 0)),
        pl.BlockSpec((NC, _RB, C), lambda i: (0, i, 0)),
        pl.BlockSpec((_RB, 1), lambda i: (i, 0)),
        pl.BlockSpec((_RB, 1), lambda i: (i, 0)),
    ],
    out_specs=pl.BlockSpec((_RB, C), lambda i: (i, 0)),
    out_shape=jax.ShapeDtypeStruct((N, C), jnp.float32),
)


def _upd_final_body(x_ref, agg_ref, d0_ref, d1_ref, out_ref):
    agg = agg_ref[0] + agg_ref[1]
    deg = jnp.maximum(d0_ref[...] + d1_ref[...], 1.0)
    out_ref[...] = x_ref[...] - agg / deg


_upd_final = pl.pallas_call(
    _upd_final_body,
    grid=(N // _RB,),
    in_specs=[
        pl.BlockSpec((_RB, C), lambda i: (i, 0)),
        pl.BlockSpec((NC, _RB, C), lambda i: (0, i, 0)),
        pl.BlockSpec((_RB, 1), lambda i: (i, 0)),
        pl.BlockSpec((_RB, 1), lambda i: (i, 0)),
    ],
    out_specs=pl.BlockSpec((_RB, C), lambda i: (i, 0)),
    out_shape=jax.ShapeDtypeStruct((N, C), jnp.float32),
)


def _sc_propagate_body(q_hbm, w_hbm, src_hbm, dst_hbm, zrow_hbm, zdeg_hbm,
                       agg_out, deg_out,
                       agg_sh, deg_sh, src_v, dst_v, ones_v,
                       q_rows, w_rows, msg, sem):
    cid = lax.axis_index("c")
    sid = lax.axis_index("s")

    r0 = sid * RSTAGE

    @pl.when(sid < NS - 1)
    def _():
        pltpu.sync_copy(zrow_hbm.at[pl.ds(r0, RSTAGE)],
                        agg_sh.at[pl.ds(r0, RSTAGE)])

    @pl.when(sid == NS - 1)
    def _():
        pltpu.sync_copy(zrow_hbm.at[pl.ds(r0, RSTAGE_LAST)],
                        agg_sh.at[pl.ds(r0, RSTAGE_LAST)])

    d0 = sid * DROWS
    pltpu.sync_copy(zdeg_hbm.at[pl.ds(d0, DROWS)], deg_sh.at[pl.ds(d0, DROWS)])
    for j in range(CHUNK // 16):
        ones_v[pl.ds(j * 16, 16)] = jnp.full((16,), 1.0, jnp.float32)
    plsc.subcore_barrier()

    nch = jnp.where(sid < EXTRA_CH, BASE_CH + 1, BASE_CH)
    c0 = cid * CORE_CHUNKS + sid * BASE_CH + jnp.minimum(sid, EXTRA_CH)

    def chunk_body(i, carry):
        base = (c0 + i) * CHUNK
        pltpu.sync_copy(src_hbm.at[pl.ds(base, CHUNK)], src_v)
        pltpu.sync_copy(dst_hbm.at[pl.ds(base, CHUNK)], dst_v)
        pltpu.sync_copy(w_hbm.at[pl.ds(base, CHUNK)], w_rows)
        pltpu.async_copy(q_hbm.at[src_v], q_rows, sem).wait()

        def mul_body(e, c2):
            for j in range(C // 16):
                sl = pl.ds(j * 16, 16)
                msg[e, sl] = q_rows[e, sl] * w_rows[e, sl]
            return c2

        lax.fori_loop(0, CHUNK, mul_body, 0)
        pltpu.sync_copy(msg, agg_sh.at[dst_v], add=True)
        pltpu.sync_copy(ones_v, deg_sh.at[dst_v], add=True)
        return carry

    lax.fori_loop(0, nch, chunk_body, 0)
    plsc.subcore_barrier()

    @pl.when(sid < NS - 1)
    def _():
        pltpu.sync_copy(agg_sh.at[pl.ds(r0, RSTAGE)],
                        agg_out.at[cid, pl.ds(r0, RSTAGE)])

    @pl.when(sid == NS - 1)
    def _():
        pltpu.sync_copy(agg_sh.at[pl.ds(r0, RSTAGE_LAST)],
                        agg_out.at[cid, pl.ds(r0, RSTAGE_LAST)])

    pltpu.sync_copy(deg_sh.at[pl.ds(d0, DROWS)],
                    deg_out.at[pl.ds(cid * NPAD + d0, DROWS)])


@functools.lru_cache(maxsize=1)
def _make_sc_propagate():
    mesh = plsc.VectorSubcoreMesh(
        core_axis_name="c", subcore_axis_name="s",
        num_cores=NC, num_subcores=NS)
    return pl.kernel(
        _sc_propagate_body,
        out_type=[
            jax.ShapeDtypeStruct((NC, N, C), jnp.float32),
            jax.ShapeDtypeStruct((NC * NPAD,), jnp.float32),
        ],
        mesh=mesh,
        scratch_types=[
            pltpu.VMEM_SHARED((N, C), jnp.float32),
            pltpu.VMEM_SHARED((NPAD,), jnp.float32),
            pltpu.VMEM((CHUNK,), jnp.int32),
            pltpu.VMEM((CHUNK,), jnp.int32),
            pltpu.VMEM((CHUNK,), jnp.float32),
            pltpu.VMEM((CHUNK, C), jnp.float32),
            pltpu.VMEM((CHUNK, C), jnp.float32),
            pltpu.VMEM((CHUNK, C), jnp.float32),
            pltpu.SemaphoreType.DMA,
        ],
    )


def kernel(input, edge_index, edge_attr, W1, b1, W2, b2):
    src = edge_index[0]
    dst = edge_index[1]
    w = _mlp(edge_attr, W1, b1.reshape(1, H), W2, b2.reshape(1, C))
    zrow = jnp.zeros((N, C), jnp.float32)
    zdeg = jnp.zeros((NPAD,), jnp.float32)

    sc_propagate = _make_sc_propagate()
    q = _softmax(input)
    agg, deg = sc_propagate(q, w, src, dst, zrow, zdeg)
    d0 = deg[:N].reshape(N, 1)
    d1 = deg[NPAD:NPAD + N].reshape(N, 1)
    q = _upd_softmax(input, agg, d0, d1)
    agg, _ = sc_propagate(q, w, src, dst, zrow, zdeg)
    return _upd_final(input, agg, d0, d1)
